# baseline (speedup 1.0000x reference)
import functools

import jax
import jax.numpy as jnp
from jax import lax
from jax.experimental import pallas as pl
from jax.experimental.pallas import tpu as pltpu
from jax.experimental.pallas import tpu_sc as plsc

_NB = 10
_A = 0.1
_GROUP = 7 * 8
_MASK_E = 0o0707070707

_NW = 32
_L = 16
_CHUNK = 32768
_SC_ROWS = 2560


def _sc_mesh():
    return plsc.VectorSubcoreMesh(core_axis_name="c", subcore_axis_name="s")


def _sc_minmax_body(in_hbm, out_hbm, buf0, buf1, mnv, mxv, sem0, sem1, *,
                    per_worker):
    wid = lax.axis_index("s") * 2 + lax.axis_index("c")
    base = wid * per_worker
    nchunks = per_worker // _CHUNK
    bufs = (buf0, buf1)
    sems = (sem0, sem1)
    handles = {}
    handles[0] = pltpu.async_copy(in_hbm.at[pl.ds(base, _CHUNK)], buf0, sem0)
    mn = jnp.full((_L,), jnp.inf, jnp.float32)
    mx = jnp.full((_L,), -jnp.inf, jnp.float32)
    for k in range(nchunks):
        if k + 1 < nchunks:
            handles[k + 1] = pltpu.async_copy(
                in_hbm.at[pl.ds(base + (k + 1) * _CHUNK, _CHUNK)],
                bufs[(k + 1) % 2], sems[(k + 1) % 2])
        handles[k].wait()
        buf = bufs[k % 2]

        def body(i, carry, buf=buf):
            mn, mx = carry
            for j in range(16):
                v = buf[pl.ds(i * 256 + j * 16, _L)]
                mn = jnp.minimum(mn, v)
                mx = jnp.maximum(mx, v)
            return mn, mx

        mn, mx = lax.fori_loop(0, _CHUNK // 256, body, (mn, mx))
    mnv[...] = mn
    mxv[...] = mx
    pltpu.sync_copy(mnv, out_hbm.at[pl.ds(wid * _L, _L)])
    pltpu.sync_copy(mxv, out_hbm.at[pl.ds(_NW * _L + wid * _L, _L)])


def _sc_minmax(flat):
    per_worker = flat.shape[0] // _NW
    run = functools.partial(
        pl.kernel,
        mesh=_sc_mesh(),
        compiler_params=pltpu.CompilerParams(needs_layout_passes=False),
        out_type=jax.ShapeDtypeStruct((2 * _NW * _L,), jnp.float32),
        scratch_types=[
            pltpu.VMEM((_CHUNK,), jnp.float32),
            pltpu.VMEM((_CHUNK,), jnp.float32),
            pltpu.VMEM((_L,), jnp.float32),
            pltpu.VMEM((_L,), jnp.float32),
            pltpu.SemaphoreType.DMA,
            pltpu.SemaphoreType.DMA,
        ],
    )(functools.partial(_sc_minmax_body, per_worker=per_worker))
    return run(flat)


def _sc_hist_body(in_hbm, pm_hbm, out_hbm, buf0, buf1, pmv, hist, outv,
                  sem0, sem1, *, start, per_worker):
    wid = lax.axis_index("s") * 2 + lax.axis_index("c")
    base = start + wid * per_worker
    nchunks = per_worker // _CHUNK
    bufs = (buf0, buf1)
    sems = (sem0, sem1)

    pltpu.sync_copy(pm_hbm, pmv)
    mn = pmv[pl.ds(0, _L)]
    mx = pmv[pl.ds(_NW * _L, _L)]
    for k in range(1, _NW):
        mn = jnp.minimum(mn, pmv[pl.ds(k * _L, _L)])
        mx = jnp.maximum(mx, pmv[pl.ds(_NW * _L + k * _L, _L)])
    lo = mn[0]
    hi = mx[0]
    for j in range(1, _L):
        lo = jnp.minimum(lo, mn[j])
        hi = jnp.maximum(hi, mx[j])
    ones_f = jnp.ones((_L,), jnp.float32)
    dv = ones_f * (hi - lo)
    seed = jnp.int32(0x7EF311C3) - lax.bitcast_convert_type(dv, jnp.int32)
    r = lax.bitcast_convert_type(seed, jnp.float32)
    for _ in range(4):
        r = r * (2.0 - dv * r)
    a = r * float(_NB)
    b = -(lo * a)

    for k in range(_NB):
        hist[pl.ds(k * _L, _L)] = jnp.zeros((_L,), jnp.int32)
    lane = lax.iota(jnp.int32, _L)
    ones = jnp.ones((_L,), jnp.int32)

    handles = {}
    handles[0] = pltpu.async_copy(in_hbm.at[pl.ds(base, _CHUNK)], buf0, sem0)
    for k in range(nchunks):
        if k + 1 < nchunks:
            handles[k + 1] = pltpu.async_copy(
                in_hbm.at[pl.ds(base + (k + 1) * _CHUNK, _CHUNK)],
                bufs[(k + 1) % 2], sems[(k + 1) % 2])
        handles[k].wait()
        buf = bufs[k % 2]

        def body(i, carry, buf=buf):
            for j in range(8):
                v = buf[pl.ds(i * 128 + j * 16, _L)]
                idx = jnp.minimum((v * a + b).astype(jnp.int32), _NB - 1)
                fidx = jnp.left_shift(idx, 4) + lane
                plsc.addupdate_scatter(hist, [fidx], ones)
            return carry

        lax.fori_loop(0, _CHUNK // 128, body, 0)

    for bb in range(_NB):
        outv[...] = hist[pl.ds(bb * _L, _L)]
        pltpu.sync_copy(outv, out_hbm.at[pl.ds(bb * _NW * _L + wid * _L, _L)])


def _sc_hist(flat, pm, start, count):
    per_worker = count // _NW
    run = functools.partial(
        pl.kernel,
        mesh=_sc_mesh(),
        compiler_params=pltpu.CompilerParams(needs_layout_passes=False),
        out_type=jax.ShapeDtypeStruct((_NB * _NW * _L,), jnp.int32),
        scratch_types=[
            pltpu.VMEM((_CHUNK,), jnp.float32),
            pltpu.VMEM((_CHUNK,), jnp.float32),
            pltpu.VMEM((2 * _NW * _L,), jnp.float32),
            pltpu.VMEM((_NB * _L,), jnp.int32),
            pltpu.VMEM((_L,), jnp.int32),
            pltpu.SemaphoreType.DMA,
            pltpu.SemaphoreType.DMA,
        ],
    )(functools.partial(_sc_hist_body, start=start, per_worker=per_worker))
    return run(flat, pm)


def _tc_mse_body(in_ref, tgt_ref, out_ref, macc_ref, *, nblk, br):
    i = pl.program_id(0)

    @pl.when(i == 0)
    def _init():
        macc_ref[...] = jnp.zeros(macc_ref.shape, macc_ref.dtype)

    m = macc_ref[...]
    for r in range(0, br, 8):
        d = in_ref[r:r + 8, :] - tgt_ref[r:r + 8, :]
        m = m + d * d
    macc_ref[...] = m

    @pl.when(i == nblk - 1)
    def _fin():
        out_ref[0] = jnp.sum(macc_ref[...])


def _tc_mse(input, target):
    rows, cols = input.shape
    br = 512
    nblk = rows // br
    return pl.pallas_call(
        functools.partial(_tc_mse_body, nblk=nblk, br=br),
        grid=(nblk,),
        in_specs=[
            pl.BlockSpec((br, cols), lambda i: (i, 0)),
            pl.BlockSpec((br, cols), lambda i: (i, 0)),
        ],
        out_specs=pl.BlockSpec(memory_space=pltpu.SMEM),
        out_shape=jax.ShapeDtypeStruct((1,), jnp.float32),
        scratch_shapes=[pltpu.VMEM((8, cols), jnp.float32)],
    )(input, target)


def _tc_hist_body(in_ref, pm_ref, hist_ref, lo_ref, hi_ref, e_ref, o_ref,
                  *, nblk, br, cols):
    i = pl.program_id(0)
    lanes = cols // 128

    @pl.when(i == 0)
    def _init():
        lo_ref[0] = jnp.min(pm_ref[0:4, :])
        hi_ref[0] = jnp.max(pm_ref[4:8, :])
        hist_ref[...] = jnp.zeros(hist_ref.shape, hist_ref.dtype)
        e_ref[...] = jnp.zeros(e_ref.shape, e_ref.dtype)
        o_ref[...] = jnp.zeros(o_ref.shape, o_ref.dtype)

    lo = lo_ref[0]
    a = _NB / (hi_ref[0] - lo)
    b = -(lo * a)
    e2 = e_ref[...]
    o2 = o_ref[...]
    r0 = 0
    while r0 < br:
        r1 = min(r0 + _GROUP, br)
        acc = jnp.zeros((8, cols), jnp.int32)
        for r in range(r0, r1, 8):
            x = in_ref[r:r + 8, :]
            idx = jnp.minimum((x * a + b).astype(jnp.int32), _NB - 1)
            acc = acc + jnp.left_shift(jnp.int32(1), idx + idx + idx)
        e2 = e2 + (acc & _MASK_E)
        o2 = o2 + (jnp.right_shift(acc, 3) & _MASK_E)
        r0 = r1
    for k in range(5):
        fe = jnp.right_shift(e2, 6 * k) & 63
        fo = jnp.right_shift(o2, 6 * k) & 63
        hist_ref[8 * (2 * k):8 * (2 * k) + 8, :] += jnp.sum(
            fe.reshape(8, lanes, 128), axis=1)
        hist_ref[8 * (2 * k + 1):8 * (2 * k + 1) + 8, :] += jnp.sum(
            fo.reshape(8, lanes, 128), axis=1)
    e_ref[...] = jnp.zeros(e_ref.shape, e_ref.dtype)
    o_ref[...] = jnp.zeros(o_ref.shape, o_ref.dtype)


def _tc_hist(input, pm2d, tc_rows):
    rows, cols = input.shape
    br = 256
    nblk = tc_rows // br
    return pl.pallas_call(
        functools.partial(_tc_hist_body, nblk=nblk, br=br, cols=cols),
        grid=(nblk,),
        in_specs=[
            pl.BlockSpec((br, cols), lambda i: (i, 0)),
            pl.BlockSpec((8, 128), lambda i: (0, 0)),
        ],
        out_specs=pl.BlockSpec((_NB * 8, 128), lambda i: (0, 0)),
        out_shape=jax.ShapeDtypeStruct((_NB * 8, 128), jnp.int32),
        scratch_shapes=[
            pltpu.SMEM((1,), jnp.float32),
            pltpu.SMEM((1,), jnp.float32),
            pltpu.VMEM((8, cols), jnp.int32),
            pltpu.VMEM((8, cols), jnp.int32),
        ],
    )(input, pm2d)


def _tc_combine_body(htc_ref, hsc_ref, mse_ref, out_ref, *, total):
    counts = []
    for bb in range(_NB):
        c_tc = jnp.sum(htc_ref[8 * bb:8 * bb + 8, :])
        c_sc = jnp.sum(hsc_ref[4 * bb:4 * bb + 4, :])
        counts.append((c_tc + c_sc).astype(jnp.float32))
    h = jnp.stack(counts) / float(total)
    entropy = -jnp.sum(h * jnp.log(h + 1e-09))
    out_ref[0] = mse_ref[0] / float(total) - _A * entropy


def _tc_combine(htc, hsc2d, mse, total):
    return pl.pallas_call(
        functools.partial(_tc_combine_body, total=total),
        in_specs=[
            pl.BlockSpec((_NB * 8, 128), lambda: (0, 0)),
            pl.BlockSpec((_NB * 4, 128), lambda: (0, 0)),
            pl.BlockSpec(memory_space=pltpu.SMEM),
        ],
        out_specs=pl.BlockSpec(memory_space=pltpu.SMEM),
        out_shape=jax.ShapeDtypeStruct((1,), jnp.float32),
    )(htc, hsc2d, mse)


def kernel(input, target):
    rows, cols = input.shape
    total = rows * cols
    flat = input.reshape(-1)
    tc_rows = rows - _SC_ROWS

    pm = _sc_minmax(flat)
    mse = _tc_mse(input, target)
    hsc = _sc_hist(flat, pm, tc_rows * cols, _SC_ROWS * cols)
    pm2d = pm.reshape(8, 128)
    htc = _tc_hist(input, pm2d, tc_rows)
    hsc2d = hsc.reshape(40, 128)
    out = _tc_combine(htc, hsc2d, mse, total)
    return out[0]


# speedup vs baseline: 2.1811x; 2.1811x over previous
import functools

import jax
import jax.numpy as jnp
from jax import lax
from jax.experimental import pallas as pl
from jax.experimental.pallas import tpu as pltpu
from jax.experimental.pallas import tpu_sc as plsc

_NB = 10
_A = 0.1
_GROUP = 7 * 8
_MASK_E = 0o0707070707

_NW = 32
_L = 16
_CROWS = 8


def _sc_minmax_body(in_hbm, out_hbm, buf0, buf1, mnv, mxv, sem0, sem1, *,
                    rows_pw, cols):
    wid = lax.axis_index("s") * 2 + lax.axis_index("c")
    base = wid * rows_pw
    nchunks = rows_pw // _CROWS
    bufs = (buf0, buf1)
    sems = (sem0, sem1)
    handles = {}
    handles[0] = pltpu.async_copy(
        in_hbm.at[pl.ds(base, _CROWS), :], buf0, sem0)
    mn = jnp.full((_L,), jnp.inf, jnp.float32)
    mx = jnp.full((_L,), -jnp.inf, jnp.float32)
    for k in range(nchunks):
        if k + 1 < nchunks:
            handles[k + 1] = pltpu.async_copy(
                in_hbm.at[pl.ds(base + (k + 1) * _CROWS, _CROWS), :],
                bufs[(k + 1) % 2], sems[(k + 1) % 2])
        handles[k].wait()
        buf = bufs[k % 2]

        def body(i, carry, buf=buf):
            mn, mx = carry
            for r in range(_CROWS):
                v = buf[r, pl.ds(i * _L, _L)]
                mn = jnp.minimum(mn, v)
                mx = jnp.maximum(mx, v)
            return mn, mx

        mn, mx = lax.fori_loop(0, cols // _L, body, (mn, mx))
    mnv[...] = mn
    mxv[...] = mx
    pltpu.sync_copy(mnv, out_hbm.at[pl.ds(wid * _L, _L)])
    pltpu.sync_copy(mxv, out_hbm.at[pl.ds(_NW * _L + wid * _L, _L)])


def _sc_minmax(input):
    rows, cols = input.shape
    run = pl.kernel(
        functools.partial(_sc_minmax_body, rows_pw=rows // _NW, cols=cols),
        mesh=plsc.VectorSubcoreMesh(core_axis_name="c", subcore_axis_name="s"),
        compiler_params=pltpu.CompilerParams(needs_layout_passes=False),
        out_type=jax.ShapeDtypeStruct((2 * _NW * _L,), jnp.float32),
        scratch_types=[
            pltpu.VMEM((_CROWS, cols), jnp.float32),
            pltpu.VMEM((_CROWS, cols), jnp.float32),
            pltpu.VMEM((_L,), jnp.float32),
            pltpu.VMEM((_L,), jnp.float32),
            pltpu.SemaphoreType.DMA,
            pltpu.SemaphoreType.DMA,
        ],
    )
    return run(input)


def _tc_mse_body(in_ref, tgt_ref, out_ref, macc_ref, *, nblk, br):
    i = pl.program_id(0)

    @pl.when(i == 0)
    def _init():
        macc_ref[...] = jnp.zeros(macc_ref.shape, macc_ref.dtype)

    m = macc_ref[...]
    for r in range(0, br, 8):
        d = in_ref[r:r + 8, :] - tgt_ref[r:r + 8, :]
        m = m + d * d
    macc_ref[...] = m

    @pl.when(i == nblk - 1)
    def _fin():
        out_ref[0] = jnp.sum(macc_ref[...])


def _tc_mse(input, target):
    rows, cols = input.shape
    br = 512
    nblk = rows // br
    return pl.pallas_call(
        functools.partial(_tc_mse_body, nblk=nblk, br=br),
        grid=(nblk,),
        in_specs=[
            pl.BlockSpec((br, cols), lambda i: (i, 0)),
            pl.BlockSpec((br, cols), lambda i: (i, 0)),
        ],
        out_specs=pl.BlockSpec(memory_space=pltpu.SMEM),
        out_shape=jax.ShapeDtypeStruct((1,), jnp.float32),
        scratch_shapes=[pltpu.VMEM((8, cols), jnp.float32)],
    )(input, target)


def _tc_hist_body(in_ref, pm_ref, mse_ref, out_ref, lo_ref, hi_ref,
                  hist_ref, e_ref, o_ref, *, nblk, br, cols, total):
    i = pl.program_id(0)
    lanes = cols // 128

    @pl.when(i == 0)
    def _init():
        lo_ref[0] = jnp.min(pm_ref[0:4, :])
        hi_ref[0] = jnp.max(pm_ref[4:8, :])
        hist_ref[...] = jnp.zeros(hist_ref.shape, hist_ref.dtype)
        e_ref[...] = jnp.zeros(e_ref.shape, e_ref.dtype)
        o_ref[...] = jnp.zeros(o_ref.shape, o_ref.dtype)

    lo = lo_ref[0]
    a = _NB / (hi_ref[0] - lo)
    b = -(lo * a)
    e2 = e_ref[...]
    o2 = o_ref[...]
    r0 = 0
    while r0 < br:
        r1 = min(r0 + _GROUP, br)
        acc = jnp.zeros((8, cols), jnp.int32)
        for r in range(r0, r1, 8):
            x = in_ref[r:r + 8, :]
            idx = jnp.minimum((x * a + b).astype(jnp.int32), _NB - 1)
            acc = acc + jnp.left_shift(jnp.int32(1), idx + idx + idx)
        e2 = e2 + (acc & _MASK_E)
        o2 = o2 + (jnp.right_shift(acc, 3) & _MASK_E)
        r0 = r1
    for k in range(5):
        fe = jnp.right_shift(e2, 6 * k) & 63
        fo = jnp.right_shift(o2, 6 * k) & 63
        hist_ref[8 * (2 * k):8 * (2 * k) + 8, :] += jnp.sum(
            fe.reshape(8, lanes, 128), axis=1)
        hist_ref[8 * (2 * k + 1):8 * (2 * k + 1) + 8, :] += jnp.sum(
            fo.reshape(8, lanes, 128), axis=1)
    e_ref[...] = jnp.zeros(e_ref.shape, e_ref.dtype)
    o_ref[...] = jnp.zeros(o_ref.shape, o_ref.dtype)

    @pl.when(i == nblk - 1)
    def _finish():
        counts = jnp.sum(
            hist_ref[...].reshape(_NB, 8 * 128).astype(jnp.float32), axis=1)
        h = counts / float(total)
        entropy = -jnp.sum(h * jnp.log(h + 1e-09))
        out_ref[0] = mse_ref[0] / float(total) - _A * entropy


def _tc_hist(input, pm2d, mse):
    rows, cols = input.shape
    br = 256
    nblk = rows // br
    total = rows * cols
    return pl.pallas_call(
        functools.partial(_tc_hist_body, nblk=nblk, br=br, cols=cols,
                          total=total),
        grid=(nblk,),
        in_specs=[
            pl.BlockSpec((br, cols), lambda i: (i, 0)),
            pl.BlockSpec((8, 128), lambda i: (0, 0)),
            pl.BlockSpec(memory_space=pltpu.SMEM),
        ],
        out_specs=pl.BlockSpec(memory_space=pltpu.SMEM),
        out_shape=jax.ShapeDtypeStruct((1,), jnp.float32),
        scratch_shapes=[
            pltpu.SMEM((1,), jnp.float32),
            pltpu.SMEM((1,), jnp.float32),
            pltpu.VMEM((_NB * 8, 128), jnp.int32),
            pltpu.VMEM((8, cols), jnp.int32),
            pltpu.VMEM((8, cols), jnp.int32),
        ],
    )(input, pm2d, mse)


def kernel(input, target):
    pm = _sc_minmax(input)
    mse = _tc_mse(input, target)
    out = _tc_hist(input, pm.reshape(8, 128), mse)
    return out[0]


# speedup vs baseline: 2.8448x; 1.3043x over previous
import functools

import jax
import jax.numpy as jnp
from jax.experimental import pallas as pl
from jax.experimental.pallas import tpu as pltpu

_NB = 10
_A = 0.1
_GROUP = 7 * 8
_MASK_E = 0o0707070707


def _tc_mse_body(in_ref, tgt_ref, mse_ref, lohi_ref, macc_ref, mn_ref, mx_ref,
                 *, nblk, br):
    i = pl.program_id(0)

    @pl.when(i == 0)
    def _init():
        macc_ref[...] = jnp.zeros(macc_ref.shape, macc_ref.dtype)
        mn_ref[...] = jnp.full(mn_ref.shape, jnp.inf, mn_ref.dtype)
        mx_ref[...] = jnp.full(mx_ref.shape, -jnp.inf, mx_ref.dtype)

    m = macc_ref[...]
    mn = mn_ref[...]
    mx = mx_ref[...]
    for r in range(0, br, 8):
        x = in_ref[r:r + 8, :]
        d = x - tgt_ref[r:r + 8, :]
        m = m + d * d
        mn = jnp.minimum(mn, x)
        mx = jnp.maximum(mx, x)
    macc_ref[...] = m
    mn_ref[...] = mn
    mx_ref[...] = mx

    @pl.when(i == nblk - 1)
    def _fin():
        mse_ref[0] = jnp.sum(macc_ref[...])
        lohi_ref[0] = jnp.min(mn_ref[...])
        lohi_ref[1] = jnp.max(mx_ref[...])


def _tc_mse(input, target):
    rows, cols = input.shape
    br = 512
    nblk = rows // br
    return pl.pallas_call(
        functools.partial(_tc_mse_body, nblk=nblk, br=br),
        grid=(nblk,),
        in_specs=[
            pl.BlockSpec((br, cols), lambda i: (i, 0)),
            pl.BlockSpec((br, cols), lambda i: (i, 0)),
        ],
        out_specs=[
            pl.BlockSpec(memory_space=pltpu.SMEM),
            pl.BlockSpec(memory_space=pltpu.SMEM),
        ],
        out_shape=[
            jax.ShapeDtypeStruct((1,), jnp.float32),
            jax.ShapeDtypeStruct((2,), jnp.float32),
        ],
        scratch_shapes=[
            pltpu.VMEM((8, cols), jnp.float32),
            pltpu.VMEM((8, cols), jnp.float32),
            pltpu.VMEM((8, cols), jnp.float32),
        ],
    )(input, target)


def _tc_hist_body(in_ref, lohi_ref, mse_ref, out_ref, hist_ref, e_ref, o_ref,
                  *, nblk, br, cols, total):
    i = pl.program_id(0)
    lanes = cols // 128

    @pl.when(i == 0)
    def _init():
        hist_ref[...] = jnp.zeros(hist_ref.shape, hist_ref.dtype)
        e_ref[...] = jnp.zeros(e_ref.shape, e_ref.dtype)
        o_ref[...] = jnp.zeros(o_ref.shape, o_ref.dtype)

    lo = lohi_ref[0]
    a = _NB / (lohi_ref[1] - lo)
    b = -(lo * a)
    e2 = e_ref[...]
    o2 = o_ref[...]
    r0 = 0
    while r0 < br:
        r1 = min(r0 + _GROUP, br)
        acc = jnp.zeros((8, cols), jnp.int32)
        for r in range(r0, r1, 8):
            x = in_ref[r:r + 8, :]
            idx = jnp.minimum((x * a + b).astype(jnp.int32), _NB - 1)
            acc = acc + jnp.left_shift(jnp.int32(1), idx + idx + idx)
        e2 = e2 + (acc & _MASK_E)
        o2 = o2 + (jnp.right_shift(acc, 3) & _MASK_E)
        r0 = r1
    for k in range(5):
        fe = jnp.right_shift(e2, 6 * k) & 63
        fo = jnp.right_shift(o2, 6 * k) & 63
        hist_ref[8 * (2 * k):8 * (2 * k) + 8, :] += jnp.sum(
            fe.reshape(8, lanes, 128), axis=1)
        hist_ref[8 * (2 * k + 1):8 * (2 * k + 1) + 8, :] += jnp.sum(
            fo.reshape(8, lanes, 128), axis=1)
    e_ref[...] = jnp.zeros(e_ref.shape, e_ref.dtype)
    o_ref[...] = jnp.zeros(o_ref.shape, o_ref.dtype)

    @pl.when(i == nblk - 1)
    def _finish():
        counts = jnp.sum(
            hist_ref[...].reshape(_NB, 8 * 128).astype(jnp.float32), axis=1)
        h = counts / float(total)
        entropy = -jnp.sum(h * jnp.log(h + 1e-09))
        out_ref[0] = mse_ref[0] / float(total) - _A * entropy


def _tc_hist(input, lohi, mse):
    rows, cols = input.shape
    br = 256
    nblk = rows // br
    total = rows * cols
    return pl.pallas_call(
        functools.partial(_tc_hist_body, nblk=nblk, br=br, cols=cols,
                          total=total),
        grid=(nblk,),
        in_specs=[
            pl.BlockSpec((br, cols), lambda i: (i, 0)),
            pl.BlockSpec(memory_space=pltpu.SMEM),
            pl.BlockSpec(memory_space=pltpu.SMEM),
        ],
        out_specs=pl.BlockSpec(memory_space=pltpu.SMEM),
        out_shape=jax.ShapeDtypeStruct((1,), jnp.float32),
        scratch_shapes=[
            pltpu.VMEM((_NB * 8, 128), jnp.int32),
            pltpu.VMEM((8, cols), jnp.int32),
            pltpu.VMEM((8, cols), jnp.int32),
        ],
    )(input, lohi, mse)


def kernel(input, target):
    mse, lohi = _tc_mse(input, target)
    out = _tc_hist(input, lohi, mse)
    return out[0]


# speedup vs baseline: 3.0156x; 1.0601x over previous
import functools

import jax
import jax.numpy as jnp
from jax.experimental import pallas as pl
from jax.experimental.pallas import tpu as pltpu

_NB = 10
_A = 0.1
_GROUP = 7 * 8
_MASK_E = 0o0707070707
_CH = 1024


def _loss_kernel(in_ref, tgt_ref, out_ref, mse_ref, lo_ref, hi_ref,
                 macc_ref, mnacc_ref, mxacc_ref, hist_ref,
                 *, nblk, br, cols, total):
    p = pl.program_id(0)
    i = pl.program_id(1)

    @pl.when((p == 0) & (i == 0))
    def _init():
        macc_ref[...] = jnp.zeros(macc_ref.shape, macc_ref.dtype)
        mnacc_ref[...] = jnp.full(mnacc_ref.shape, jnp.inf, mnacc_ref.dtype)
        mxacc_ref[...] = jnp.full(mxacc_ref.shape, -jnp.inf, mxacc_ref.dtype)
        hist_ref[...] = jnp.zeros(hist_ref.shape, hist_ref.dtype)

    @pl.when(p == 0)
    def _pass0():
        for c0 in range(0, cols, _CH):
            m = macc_ref[:, c0:c0 + _CH]
            mn = mnacc_ref[:, c0:c0 + _CH]
            mx = mxacc_ref[:, c0:c0 + _CH]
            for r in range(0, br, 8):
                x = in_ref[r:r + 8, c0:c0 + _CH]
                d = x - tgt_ref[r:r + 8, c0:c0 + _CH]
                m = m + d * d
                mn = jnp.minimum(mn, x)
                mx = jnp.maximum(mx, x)
            macc_ref[:, c0:c0 + _CH] = m
            mnacc_ref[:, c0:c0 + _CH] = mn
            mxacc_ref[:, c0:c0 + _CH] = mx

    @pl.when((p == 0) & (i == nblk - 1))
    def _minmax():
        mse_ref[0] = jnp.sum(macc_ref[...])
        lo_ref[0] = jnp.min(mnacc_ref[...])
        hi_ref[0] = jnp.max(mxacc_ref[...])

    @pl.when(p == 1)
    def _pass1():
        lo = lo_ref[0]
        a = _NB / (hi_ref[0] - lo)
        b = -(lo * a)
        lanes_c = _CH // 128
        for c0 in range(0, cols, _CH):
            e2 = jnp.zeros((8, _CH), jnp.int32)
            o2 = jnp.zeros((8, _CH), jnp.int32)
            r0 = 0
            while r0 < br:
                r1 = min(r0 + _GROUP, br)
                acc = jnp.zeros((8, _CH), jnp.int32)
                for r in range(r0, r1, 8):
                    x = in_ref[r:r + 8, c0:c0 + _CH]
                    idx = jnp.minimum((x * a + b).astype(jnp.int32), _NB - 1)
                    acc = acc + jnp.left_shift(jnp.int32(1), idx + idx + idx)
                e2 = e2 + (acc & _MASK_E)
                o2 = o2 + (jnp.right_shift(acc, 3) & _MASK_E)
                r0 = r1
            for k in range(5):
                fe = jnp.right_shift(e2, 6 * k) & 63
                fo = jnp.right_shift(o2, 6 * k) & 63
                hist_ref[8 * (2 * k):8 * (2 * k) + 8, :] += jnp.sum(
                    fe.reshape(8, lanes_c, 128), axis=1)
                hist_ref[8 * (2 * k + 1):8 * (2 * k + 1) + 8, :] += jnp.sum(
                    fo.reshape(8, lanes_c, 128), axis=1)

    @pl.when((p == 1) & (i == nblk - 1))
    def _finish():
        counts = jnp.sum(
            hist_ref[...].reshape(_NB, 8 * 128).astype(jnp.float32), axis=1)
        h = counts / float(total)
        entropy = -jnp.sum(h * jnp.log(h + 1e-09))
        out_ref[0] = mse_ref[0] / float(total) - _A * entropy


def kernel(input, target):
    rows, cols = input.shape
    br = 256 if rows % 256 == 0 else rows
    nblk = rows // br
    total = rows * cols
    assert br // 8 <= 63

    out = pl.pallas_call(
        functools.partial(_loss_kernel, nblk=nblk, br=br, cols=cols,
                          total=total),
        grid=(2, nblk),
        in_specs=[
            pl.BlockSpec((br, cols), lambda p, i: (i, 0)),
            pl.BlockSpec((br, cols), lambda p, i: (i * (1 - p), 0)),
        ],
        out_specs=pl.BlockSpec(memory_space=pltpu.SMEM),
        out_shape=jax.ShapeDtypeStruct((1,), jnp.float32),
        scratch_shapes=[
            pltpu.SMEM((1,), jnp.float32),
            pltpu.SMEM((1,), jnp.float32),
            pltpu.SMEM((1,), jnp.float32),
            pltpu.VMEM((8, cols), jnp.float32),
            pltpu.VMEM((8, cols), jnp.float32),
            pltpu.VMEM((8, cols), jnp.float32),
            pltpu.VMEM((_NB * 8, 128), jnp.int32),
        ],
    )(input, target)
    return out[0]
